# baseline (speedup 1.0000x reference)
import functools

import jax
import jax.numpy as jnp
import numpy as np
from jax import lax
from jax.experimental import pallas as pl
from jax.experimental.pallas import tpu as pltpu
import jax.experimental.pallas.tpu_sc as plsc


def _floor_i32(v):
    t = v.astype(jnp.int32)
    tf = t.astype(jnp.float32)
    return jnp.where(tf > v, t - 1, t)


def kernel(features, grid):
    B, H, W, C = features.shape
    P = grid.shape[1]
    N = B * P
    L = 16

    feat_bf = features.reshape(B * H * W, C).astype(jnp.bfloat16)
    feat = jax.lax.bitcast_convert_type(
        feat_bf.reshape(B * H * W, C // 2, 2), jnp.int32)
    gy = grid[..., 1].reshape(N).astype(jnp.float32)
    gx = grid[..., 0].reshape(N).astype(jnp.float32)

    mesh = plsc.VectorSubcoreMesh(core_axis_name="c", subcore_axis_name="s")
    NW = mesh.num_cores * mesh.num_subcores
    n_per_w = N // NW
    PTS = 32
    n_it = n_per_w // PTS
    NB = 2

    @functools.partial(
        pl.kernel,
        mesh=mesh,
        out_type=jax.ShapeDtypeStruct((N, C), jnp.float32),
        scratch_types=[
            pltpu.VMEM((n_per_w,), jnp.float32),
            pltpu.VMEM((n_per_w,), jnp.float32),
            [[pltpu.VMEM((PTS,), jnp.int32) for _ in range(4)]
             for _ in range(NB)],
            [[pltpu.VMEM((PTS + L,), jnp.float32) for _ in range(4)]
             for _ in range(NB)],
            [[pltpu.VMEM((PTS, C // 2), jnp.int32) for _ in range(4)]
             for _ in range(NB)],
            [pltpu.VMEM((PTS, C), jnp.float32) for _ in range(NB)],
            [pltpu.SemaphoreType.DMA for _ in range(NB)],
            [pltpu.SemaphoreType.DMA for _ in range(NB)],
        ],
    )
    def run(feat_hbm, gy_hbm, gx_hbm, out_hbm,
            gy_v, gx_v, idx_vs, w_vs, row_vs, ob_vs, gsems, osems):
        cid = lax.axis_index("c")
        sid = lax.axis_index("s")
        wid = sid * mesh.num_cores + cid
        base = wid * n_per_w
        boff = (base // P) * (H * W)

        pltpu.sync_copy(gy_hbm.at[pl.ds(base, n_per_w)], gy_v)
        pltpu.sync_copy(gx_hbm.at[pl.ds(base, n_per_w)], gx_v)

        corners = ((0, 0), (1, 0), (0, 1), (1, 1))

        def fire(it, s):
            for sub in range(PTS // L):
                off = it * PTS + sub * L
                y = gy_v[pl.ds(off, L)] * float(H) - 0.5
                x = gx_v[pl.ds(off, L)] * float(W) - 0.5
                yi = _floor_i32(y)
                xi = _floor_i32(x)
                fy = y - yi.astype(jnp.float32)
                fx = x - xi.astype(jnp.float32)
                wy = (1.0 - fy, fy)
                wx = (1.0 - fx, fx)
                for ci, (dy, dx) in enumerate(corners):
                    yc = yi + dy
                    xc = xi + dx
                    valid = ((yc >= 0) & (yc < H) & (xc >= 0) & (xc < W))
                    ycl = jnp.clip(yc, 0, H - 1)
                    xcl = jnp.clip(xc, 0, W - 1)
                    idx_vs[s][ci][pl.ds(sub * L, L)] = boff + ycl * W + xcl
                    w = wy[dy] * wx[dx]
                    w_vs[s][ci][pl.ds(sub * L, L)] = jnp.where(valid, w, 0.0)
            for ci in range(4):
                pltpu.async_copy(feat_hbm.at[idx_vs[s][ci]], row_vs[s][ci],
                                 gsems[s])

        def consume(it, s, first):
            for ci in range(4):
                pltpu.make_async_copy(feat_hbm.at[idx_vs[s][ci]],
                                      row_vs[s][ci], gsems[s]).wait()
            if not first:
                pltpu.make_async_copy(
                    ob_vs[s], out_hbm.at[pl.ds(base, PTS)], osems[s]).wait()

            hi_mask = jnp.int32(-65536)
            iota = jnp.arange(L, dtype=jnp.int32)
            il = iota >> 1
            ih = il + L // 2
            even_lane = (iota & 1) == 0

            def _take(v, idx):
                return v.at[idx].get(mode="promise_in_bounds")

            def pt_body(j, c2):
                ws = [w_vs[s][ci][pl.ds(j, L)][0] for ci in range(4)]
                for cb in range(C // 32):
                    sl = pl.ds(cb * L, L)
                    acc_e = jnp.zeros((L,), jnp.float32)
                    acc_o = jnp.zeros((L,), jnp.float32)
                    for ci in range(4):
                        b = row_vs[s][ci][j, sl]
                        ev = lax.bitcast_convert_type(b << 16, jnp.float32)
                        od = lax.bitcast_convert_type(b & hi_mask, jnp.float32)
                        acc_e = acc_e + ws[ci] * ev
                        acc_o = acc_o + ws[ci] * od
                    lo = jnp.where(even_lane, _take(acc_e, il), _take(acc_o, il))
                    hi = jnp.where(even_lane, _take(acc_e, ih), _take(acc_o, ih))
                    ob_vs[s][j, pl.ds(cb * 32, L)] = lo
                    ob_vs[s][j, pl.ds(cb * 32 + L, L)] = hi
                return c2

            lax.fori_loop(0, PTS, pt_body, 0)
            pltpu.async_copy(ob_vs[s], out_hbm.at[pl.ds(base + it * PTS, PTS)],
                             osems[s])

        fire(0, 0)
        fire(1, 1)

        def it_body(it2, carry):
            it = it2 * NB
            consume(it, 0, False)
            fire(it + 2, 0)
            consume(it + 1, 1, False)
            fire(it + 3, 1)
            return carry

        consume(0, 0, True)
        fire(2, 0)
        consume(1, 1, True)
        fire(3, 1)
        lax.fori_loop(1, n_it // NB - 1, it_body, 0)
        consume(n_it - 2, 0, False)
        consume(n_it - 1, 1, False)
        for s in range(NB):
            pltpu.make_async_copy(
                ob_vs[s], out_hbm.at[pl.ds(base, PTS)], osems[s]).wait()

    out = run(feat, gy, gx)
    return out.reshape(B, P, C).astype(features.dtype)


# speedup vs baseline: 1.2255x; 1.2255x over previous
import functools

import jax
import jax.numpy as jnp
from jax import lax
from jax.experimental import pallas as pl
from jax.experimental.pallas import tpu as pltpu
import jax.experimental.pallas.tpu_sc as plsc


def _floor_i32(v):
    t = v.astype(jnp.int32)
    tf = t.astype(jnp.float32)
    return jnp.where(tf > v, t - 1, t)


def kernel(features, grid):
    B, H, W, C = features.shape
    P = grid.shape[1]
    N = B * P
    L = 16

    feat_bf = features.reshape(B * H * W, C).astype(jnp.bfloat16)
    feat = jax.lax.bitcast_convert_type(
        feat_bf.reshape(B * H * W, C // 2, 2), jnp.int32)
    gy = grid[..., 1].reshape(N).astype(jnp.float32)
    gx = grid[..., 0].reshape(N).astype(jnp.float32)

    mesh = plsc.VectorSubcoreMesh(core_axis_name="c", subcore_axis_name="s")
    NW = mesh.num_cores * mesh.num_subcores
    n_per_w = N // NW
    PTS = 32
    n_it = n_per_w // PTS
    NB = 2

    @functools.partial(
        pl.kernel,
        mesh=mesh,
        out_type=jax.ShapeDtypeStruct((N, C), jnp.float32),
        scratch_types=[
            pltpu.VMEM((n_per_w,), jnp.float32),
            pltpu.VMEM((n_per_w,), jnp.float32),
            [pltpu.VMEM((4 * PTS,), jnp.int32) for _ in range(NB)],
            [[pltpu.VMEM((PTS + L,), jnp.float32) for _ in range(4)]
             for _ in range(NB)],
            [pltpu.VMEM((4 * PTS, C // 2), jnp.int32)
             for _ in range(NB)],
            [pltpu.VMEM((PTS, C), jnp.float32) for _ in range(NB)],
            [pltpu.SemaphoreType.DMA for _ in range(NB)],
            [pltpu.SemaphoreType.DMA for _ in range(NB)],
        ],
    )
    def run(feat_hbm, gy_hbm, gx_hbm, out_hbm,
            gy_v, gx_v, idx_vs, w_vs, row_vs, ob_vs, gsems, osems):
        cid = lax.axis_index("c")
        sid = lax.axis_index("s")
        wid = sid * mesh.num_cores + cid
        base = wid * n_per_w
        boff = (base // P) * (H * W)

        pltpu.sync_copy(gy_hbm.at[pl.ds(base, n_per_w)], gy_v)
        pltpu.sync_copy(gx_hbm.at[pl.ds(base, n_per_w)], gx_v)

        corners = ((0, 0), (1, 0), (0, 1), (1, 1))

        def fire(it, s):
            for sub in range(PTS // L):
                off = it * PTS + sub * L
                y = gy_v[pl.ds(off, L)] * float(H) - 0.5
                x = gx_v[pl.ds(off, L)] * float(W) - 0.5
                yi = _floor_i32(y)
                xi = _floor_i32(x)
                fy = y - yi.astype(jnp.float32)
                fx = x - xi.astype(jnp.float32)
                wy = (1.0 - fy, fy)
                wx = (1.0 - fx, fx)
                for ci, (dy, dx) in enumerate(corners):
                    yc = yi + dy
                    xc = xi + dx
                    valid = ((yc >= 0) & (yc < H) & (xc >= 0) & (xc < W))
                    ycl = jnp.clip(yc, 0, H - 1)
                    xcl = jnp.clip(xc, 0, W - 1)
                    idx_vs[s][pl.ds(ci * PTS + sub * L, L)] = (
                        boff + ycl * W + xcl)
                    w = wy[dy] * wx[dx]
                    w_vs[s][ci][pl.ds(sub * L, L)] = jnp.where(valid, w, 0.0)
            pltpu.async_copy(feat_hbm.at[idx_vs[s]], row_vs[s], gsems[s])

        hi_mask = jnp.int32(-65536)
        iota = jnp.arange(L, dtype=jnp.int32)
        il = iota >> 1
        ih = il + L // 2
        even_lane = (iota & 1) == 0

        def _take(v, idx):
            return v.at[idx].get(mode="promise_in_bounds")

        def consume(it, s, first):
            pltpu.make_async_copy(feat_hbm.at[idx_vs[s]], row_vs[s],
                                  gsems[s]).wait()
            if not first:
                pltpu.make_async_copy(
                    ob_vs[s], out_hbm.at[pl.ds(base, PTS)], osems[s]).wait()

            @plsc.parallel_loop(0, PTS, 1, unroll=2)
            def pt_body(j):
                ws = [w_vs[s][ci][pl.ds(j, L)][0] for ci in range(4)]
                for cb in range(C // 32):
                    sl = pl.ds(cb * L, L)
                    es, os_ = [], []
                    for ci in range(4):
                        b = row_vs[s][ci * PTS + j, sl]
                        es.append(lax.bitcast_convert_type(b << 16,
                                                           jnp.float32))
                        os_.append(lax.bitcast_convert_type(b & hi_mask,
                                                            jnp.float32))
                    acc_e = ((ws[0] * es[0] + ws[1] * es[1])
                             + (ws[2] * es[2] + ws[3] * es[3]))
                    acc_o = ((ws[0] * os_[0] + ws[1] * os_[1])
                             + (ws[2] * os_[2] + ws[3] * os_[3]))
                    lo = jnp.where(even_lane, _take(acc_e, il), _take(acc_o, il))
                    hi = jnp.where(even_lane, _take(acc_e, ih), _take(acc_o, ih))
                    ob_vs[s][j, pl.ds(cb * 32, L)] = lo
                    ob_vs[s][j, pl.ds(cb * 32 + L, L)] = hi

            pltpu.async_copy(ob_vs[s], out_hbm.at[pl.ds(base + it * PTS, PTS)],
                             osems[s])

        fire(0, 0)
        fire(1, 1)

        def it_body(it2, carry):
            it = it2 * NB
            consume(it, 0, False)
            fire(it + 2, 0)
            consume(it + 1, 1, False)
            fire(it + 3, 1)
            return carry

        consume(0, 0, True)
        fire(2, 0)
        consume(1, 1, True)
        fire(3, 1)
        lax.fori_loop(1, n_it // NB - 1, it_body, 0)
        consume(n_it - 2, 0, False)
        consume(n_it - 1, 1, False)
        for s in range(NB):
            pltpu.make_async_copy(
                ob_vs[s], out_hbm.at[pl.ds(base, PTS)], osems[s]).wait()

    out = run(feat, gy, gx)
    return out.reshape(B, P, C).astype(features.dtype)


# speedup vs baseline: 2.6401x; 2.1543x over previous
import functools

import jax
import jax.numpy as jnp
from jax import lax
from jax.experimental import pallas as pl
from jax.experimental.pallas import tpu as pltpu
import jax.experimental.pallas.tpu_sc as plsc


def _floor_i32(v):
    t = v.astype(jnp.int32)
    tf = t.astype(jnp.float32)
    return jnp.where(tf > v, t - 1, t)


def kernel(features, grid):
    B, H, W, C = features.shape
    P = grid.shape[1]
    N = B * P
    L = 16

    feat = features.reshape(B * H * W, C).astype(jnp.float32)
    gy = grid[..., 1].reshape(N).astype(jnp.float32)
    gx = grid[..., 0].reshape(N).astype(jnp.float32)

    mesh = plsc.VectorSubcoreMesh(core_axis_name="c", subcore_axis_name="s")
    NW = mesh.num_cores * mesh.num_subcores
    n_per_w = N // NW
    PTS = 32
    n_it = n_per_w // PTS
    NB = 2

    @functools.partial(
        pl.kernel,
        mesh=mesh,
        out_type=jax.ShapeDtypeStruct((N, C), jnp.float32),
        scratch_types=[
            pltpu.VMEM((n_per_w,), jnp.float32),
            pltpu.VMEM((n_per_w,), jnp.float32),
            [pltpu.VMEM((4 * PTS,), jnp.int32) for _ in range(NB)],
            [[pltpu.VMEM((PTS + L,), jnp.float32) for _ in range(4)]
             for _ in range(NB)],
            [pltpu.VMEM((4 * PTS, C), jnp.float32)
             for _ in range(NB)],
            [pltpu.VMEM((PTS, C), jnp.float32) for _ in range(NB)],
            [pltpu.SemaphoreType.DMA for _ in range(NB)],
            [pltpu.SemaphoreType.DMA for _ in range(NB)],
        ],
    )
    def run(feat_hbm, gy_hbm, gx_hbm, out_hbm,
            gy_v, gx_v, idx_vs, w_vs, row_vs, ob_vs, gsems, osems):
        cid = lax.axis_index("c")
        sid = lax.axis_index("s")
        wid = sid * mesh.num_cores + cid
        base = wid * n_per_w
        boff = (base // P) * (H * W)

        pltpu.sync_copy(gy_hbm.at[pl.ds(base, n_per_w)], gy_v)
        pltpu.sync_copy(gx_hbm.at[pl.ds(base, n_per_w)], gx_v)

        corners = ((0, 0), (1, 0), (0, 1), (1, 1))

        def fire(it, s):
            for sub in range(PTS // L):
                off = it * PTS + sub * L
                y = gy_v[pl.ds(off, L)] * float(H) - 0.5
                x = gx_v[pl.ds(off, L)] * float(W) - 0.5
                yi = _floor_i32(y)
                xi = _floor_i32(x)
                fy = y - yi.astype(jnp.float32)
                fx = x - xi.astype(jnp.float32)
                wy = (1.0 - fy, fy)
                wx = (1.0 - fx, fx)
                for ci, (dy, dx) in enumerate(corners):
                    yc = yi + dy
                    xc = xi + dx
                    valid = ((yc >= 0) & (yc < H) & (xc >= 0) & (xc < W))
                    ycl = jnp.clip(yc, 0, H - 1)
                    xcl = jnp.clip(xc, 0, W - 1)
                    idx_vs[s][pl.ds(ci * PTS + sub * L, L)] = (
                        boff + ycl * W + xcl)
                    w = wy[dy] * wx[dx]
                    w_vs[s][ci][pl.ds(sub * L, L)] = jnp.where(valid, w, 0.0)
            pltpu.async_copy(feat_hbm.at[idx_vs[s]], row_vs[s], gsems[s])

        hi_mask = jnp.int32(-65536)
        iota = jnp.arange(L, dtype=jnp.int32)
        il = iota >> 1
        ih = il + L // 2
        even_lane = (iota & 1) == 0

        def _take(v, idx):
            return v.at[idx].get(mode="promise_in_bounds")

        def consume(it, s, first):
            pltpu.make_async_copy(feat_hbm.at[idx_vs[s]], row_vs[s],
                                  gsems[s]).wait()
            if not first:
                pltpu.make_async_copy(
                    ob_vs[s], out_hbm.at[pl.ds(base, PTS)], osems[s]).wait()

            @plsc.parallel_loop(0, PTS, 1, unroll=2)
            def pt_body(j):
                ws = [w_vs[s][ci][pl.ds(j, L)][0] for ci in range(4)]
                for cb in range(C // L):
                    sl = pl.ds(cb * L, L)
                    rs = [row_vs[s][ci * PTS + j, sl] for ci in range(4)]
                    ob_vs[s][j, sl] = ((ws[0] * rs[0] + ws[1] * rs[1])
                                       + (ws[2] * rs[2] + ws[3] * rs[3]))

            pltpu.async_copy(ob_vs[s], out_hbm.at[pl.ds(base + it * PTS, PTS)],
                             osems[s])

        fire(0, 0)
        fire(1, 1)

        def it_body(it2, carry):
            it = it2 * NB
            consume(it, 0, False)
            fire(it + 2, 0)
            consume(it + 1, 1, False)
            fire(it + 3, 1)
            return carry

        consume(0, 0, True)
        fire(2, 0)
        consume(1, 1, True)
        fire(3, 1)
        lax.fori_loop(1, n_it // NB - 1, it_body, 0)
        consume(n_it - 2, 0, False)
        consume(n_it - 1, 1, False)
        for s in range(NB):
            pltpu.make_async_copy(
                ob_vs[s], out_hbm.at[pl.ds(base, PTS)], osems[s]).wait()

    out = run(feat, gy, gx)
    return out.reshape(B, P, C).astype(features.dtype)


# speedup vs baseline: 2.8494x; 1.0793x over previous
import functools

import jax
import jax.numpy as jnp
from jax import lax
from jax.experimental import pallas as pl
from jax.experimental.pallas import tpu as pltpu
import jax.experimental.pallas.tpu_sc as plsc


def _floor_i32(v):
    t = v.astype(jnp.int32)
    tf = t.astype(jnp.float32)
    return jnp.where(tf > v, t - 1, t)


def kernel(features, grid):
    B, H, W, C = features.shape
    P = grid.shape[1]
    N = B * P
    L = 16

    feat = features.reshape(B * H * W, C).astype(jnp.float32)
    gy = grid[..., 1].reshape(N).astype(jnp.float32)
    gx = grid[..., 0].reshape(N).astype(jnp.float32)

    mesh = plsc.VectorSubcoreMesh(core_axis_name="c", subcore_axis_name="s")
    NW = mesh.num_cores * mesh.num_subcores
    n_per_w = N // NW
    PTS = 32
    n_it = n_per_w // PTS
    NB = 2

    @functools.partial(
        pl.kernel,
        mesh=mesh,
        out_type=jax.ShapeDtypeStruct((N, C), jnp.float32),
        scratch_types=[
            pltpu.VMEM((n_per_w,), jnp.float32),
            pltpu.VMEM((n_per_w,), jnp.float32),
            [[pltpu.VMEM((PTS,), jnp.int32) for _ in range(4)]
             for _ in range(NB)],
            [[pltpu.VMEM((PTS + L,), jnp.float32) for _ in range(4)]
             for _ in range(NB)],
            [[pltpu.VMEM((PTS, C), jnp.float32) for _ in range(4)]
             for _ in range(NB)],
            [pltpu.VMEM((PTS, C), jnp.float32) for _ in range(NB)],
            [pltpu.SemaphoreType.DMA for _ in range(NB)],
            [pltpu.SemaphoreType.DMA for _ in range(NB)],
        ],
    )
    def run(feat_hbm, gy_hbm, gx_hbm, out_hbm,
            gy_v, gx_v, idx_vs, w_vs, row_vs, ob_vs, gsems, osems):
        cid = lax.axis_index("c")
        sid = lax.axis_index("s")
        wid = sid * mesh.num_cores + cid
        base = wid * n_per_w
        boff = (base // P) * (H * W)

        pltpu.sync_copy(gy_hbm.at[pl.ds(base, n_per_w)], gy_v)
        pltpu.sync_copy(gx_hbm.at[pl.ds(base, n_per_w)], gx_v)

        corners = ((0, 0), (1, 0), (0, 1), (1, 1))

        def fire(it, s):
            for sub in range(PTS // L):
                off = it * PTS + sub * L
                y = gy_v[pl.ds(off, L)] * float(H) - 0.5
                x = gx_v[pl.ds(off, L)] * float(W) - 0.5
                yi = _floor_i32(y)
                xi = _floor_i32(x)
                fy = y - yi.astype(jnp.float32)
                fx = x - xi.astype(jnp.float32)
                wy = (1.0 - fy, fy)
                wx = (1.0 - fx, fx)
                for ci, (dy, dx) in enumerate(corners):
                    yc = yi + dy
                    xc = xi + dx
                    valid = ((yc >= 0) & (yc < H) & (xc >= 0) & (xc < W))
                    ycl = jnp.clip(yc, 0, H - 1)
                    xcl = jnp.clip(xc, 0, W - 1)
                    idx_vs[s][ci][pl.ds(sub * L, L)] = boff + ycl * W + xcl
                    w = wy[dy] * wx[dx]
                    w_vs[s][ci][pl.ds(sub * L, L)] = jnp.where(valid, w, 0.0)
            for ci in range(4):
                pltpu.async_copy(feat_hbm.at[idx_vs[s][ci]], row_vs[s][ci],
                                 gsems[s])

        hi_mask = jnp.int32(-65536)
        iota = jnp.arange(L, dtype=jnp.int32)
        il = iota >> 1
        ih = il + L // 2
        even_lane = (iota & 1) == 0

        def _take(v, idx):
            return v.at[idx].get(mode="promise_in_bounds")

        def consume(it, s, first):
            for ci in range(4):
                pltpu.make_async_copy(feat_hbm.at[idx_vs[s][ci]],
                                      row_vs[s][ci], gsems[s]).wait()
            if not first:
                pltpu.make_async_copy(
                    ob_vs[s], out_hbm.at[pl.ds(base, PTS)], osems[s]).wait()

            @plsc.parallel_loop(0, PTS, 1)
            def pt_body(j):
                ws = [w_vs[s][ci][pl.ds(j, L)][0] for ci in range(4)]
                for cb in range(C // L):
                    sl = pl.ds(cb * L, L)
                    rs = [row_vs[s][ci][j, sl] for ci in range(4)]
                    ob_vs[s][j, sl] = ((ws[0] * rs[0] + ws[1] * rs[1])
                                       + (ws[2] * rs[2] + ws[3] * rs[3]))

            pltpu.async_copy(ob_vs[s], out_hbm.at[pl.ds(base + it * PTS, PTS)],
                             osems[s])

        fire(0, 0)
        fire(1, 1)

        def it_body(it2, carry):
            it = it2 * NB
            consume(it, 0, False)
            fire(it + 2, 0)
            consume(it + 1, 1, False)
            fire(it + 3, 1)
            return carry

        consume(0, 0, True)
        fire(2, 0)
        consume(1, 1, True)
        fire(3, 1)
        lax.fori_loop(1, n_it // NB - 1, it_body, 0)
        consume(n_it - 2, 0, False)
        consume(n_it - 1, 1, False)
        for s in range(NB):
            pltpu.make_async_copy(
                ob_vs[s], out_hbm.at[pl.ds(base, PTS)], osems[s]).wait()

    out = run(feat, gy, gx)
    return out.reshape(B, P, C).astype(features.dtype)


# speedup vs baseline: 2.9795x; 1.0456x over previous
import functools

import jax
import jax.numpy as jnp
from jax import lax
from jax.experimental import pallas as pl
from jax.experimental.pallas import tpu as pltpu
import jax.experimental.pallas.tpu_sc as plsc


def _floor_i32(v):
    t = v.astype(jnp.int32)
    tf = t.astype(jnp.float32)
    return jnp.where(tf > v, t - 1, t)


def kernel(features, grid):
    B, H, W, C = features.shape
    P = grid.shape[1]
    N = B * P
    L = 16

    feat = features.reshape(B * H * W, C).astype(jnp.float32)
    gy = grid[..., 1].reshape(N).astype(jnp.float32)
    gx = grid[..., 0].reshape(N).astype(jnp.float32)

    mesh = plsc.VectorSubcoreMesh(core_axis_name="c", subcore_axis_name="s")
    NW = mesh.num_cores * mesh.num_subcores
    n_per_w = N // NW
    PTS = 32
    n_it = n_per_w // PTS
    NB = 2

    @functools.partial(
        pl.kernel,
        mesh=mesh,
        out_type=jax.ShapeDtypeStruct((N, C), jnp.float32),
        scratch_types=[
            pltpu.VMEM((n_per_w,), jnp.float32),
            pltpu.VMEM((n_per_w,), jnp.float32),
            [[pltpu.VMEM((PTS,), jnp.int32) for _ in range(4)]
             for _ in range(NB)],
            [pltpu.VMEM((PTS + L,), jnp.float32) for _ in range(4)],
            [[pltpu.VMEM((PTS, C), jnp.float32) for _ in range(4)]
             for _ in range(NB)],
            [pltpu.VMEM((PTS, C), jnp.float32) for _ in range(NB)],
            [pltpu.SemaphoreType.DMA for _ in range(NB)],
            [pltpu.SemaphoreType.DMA for _ in range(NB)],
        ],
    )
    def run(feat_hbm, gy_hbm, gx_hbm, out_hbm,
            gy_v, gx_v, idx_vs, w_vs, row_vs, ob_vs, gsems, osems):
        cid = lax.axis_index("c")
        sid = lax.axis_index("s")
        wid = sid * mesh.num_cores + cid
        base = wid * n_per_w
        boff = (base // P) * (H * W)

        pltpu.sync_copy(gy_hbm.at[pl.ds(base, n_per_w)], gy_v)
        pltpu.sync_copy(gx_hbm.at[pl.ds(base, n_per_w)], gx_v)

        corners = ((0, 0), (1, 0), (0, 1), (1, 1))

        def fire(it, s):
            for sub in range(PTS // L):
                off = it * PTS + sub * L
                y = gy_v[pl.ds(off, L)] * float(H) - 0.5
                x = gx_v[pl.ds(off, L)] * float(W) - 0.5
                yi = _floor_i32(y)
                xi = _floor_i32(x)
                for ci, (dy, dx) in enumerate(corners):
                    ycl = jnp.clip(yi + dy, 0, H - 1)
                    xcl = jnp.clip(xi + dx, 0, W - 1)
                    idx_vs[s][ci][pl.ds(sub * L, L)] = boff + ycl * W + xcl
            for ci in range(4):
                pltpu.async_copy(feat_hbm.at[idx_vs[s][ci]], row_vs[s][ci],
                                 gsems[s])

        def wait_gather(s):
            for ci in range(4):
                pltpu.make_async_copy(feat_hbm.at[idx_vs[s][ci]],
                                      row_vs[s][ci], gsems[s]).wait()

        def combine(it, s, first):
            for sub in range(PTS // L):
                off = it * PTS + sub * L
                y = gy_v[pl.ds(off, L)] * float(H) - 0.5
                x = gx_v[pl.ds(off, L)] * float(W) - 0.5
                yi = _floor_i32(y)
                xi = _floor_i32(x)
                fy = y - yi.astype(jnp.float32)
                fx = x - xi.astype(jnp.float32)
                wy = (1.0 - fy, fy)
                wx = (1.0 - fx, fx)
                for ci, (dy, dx) in enumerate(corners):
                    yc = yi + dy
                    xc = xi + dx
                    valid = ((yc >= 0) & (yc < H) & (xc >= 0) & (xc < W))
                    w = wy[dy] * wx[dx]
                    w_vs[ci][pl.ds(sub * L, L)] = jnp.where(valid, w, 0.0)
            if not first:
                pltpu.make_async_copy(
                    ob_vs[s], out_hbm.at[pl.ds(base, PTS)], osems[s]).wait()

            @plsc.parallel_loop(0, PTS, 1)
            def pt_body(j):
                ws = [w_vs[ci][pl.ds(j, L)][0] for ci in range(4)]
                for cb in range(C // L):
                    sl = pl.ds(cb * L, L)
                    rs = [row_vs[s][ci][j, sl] for ci in range(4)]
                    ob_vs[s][j, sl] = ((ws[0] * rs[0] + ws[1] * rs[1])
                                       + (ws[2] * rs[2] + ws[3] * rs[3]))

            pltpu.async_copy(ob_vs[s], out_hbm.at[pl.ds(base + it * PTS, PTS)],
                             osems[s])

        fire(0, 0)
        fire(1, 1)

        def it_body(it2, carry):
            it = it2 * NB
            wait_gather(0)
            fire(it + 2, 0)
            combine(it, 0, False)
            wait_gather(1)
            fire(it + 3, 1)
            combine(it + 1, 1, False)
            return carry

        wait_gather(0)
        fire(2, 0)
        combine(0, 0, True)
        wait_gather(1)
        fire(3, 1)
        combine(1, 1, True)
        lax.fori_loop(1, n_it // NB - 1, it_body, 0)
        wait_gather(0)
        combine(n_it - 2, 0, False)
        wait_gather(1)
        combine(n_it - 1, 1, False)
        for s in range(NB):
            pltpu.make_async_copy(
                ob_vs[s], out_hbm.at[pl.ds(base, PTS)], osems[s]).wait()

    out = run(feat, gy, gx)
    return out.reshape(B, P, C).astype(features.dtype)
